# Initial kernel scaffold; baseline (speedup 1.0000x reference)
import jax
import jax.numpy as jnp
from jax.experimental import pallas as pl


def kernel(x, relative_attention_bias_table):
    raise NotImplementedError("write your pallas kernel here")


# speedup vs baseline: 9.8612x; 9.8612x over previous
import functools

import jax
import jax.numpy as jnp
from jax import lax
from jax.experimental import pallas as pl
from jax.experimental.pallas import tpu as pltpu
from jax.experimental.pallas import tpu_sc as plsc

_MAX_DISTANCE = 128
_NUM_CORES = 2
_NUM_SUBCORES = 16
_LANES = 16


def _bias_body(q_len, k_len, heads, tab_hbm, out_hbm, t_vmem, n_vmem, sem):
    num_w = _NUM_CORES * _NUM_SUBCORES
    rows_per_w = q_len // num_w
    t_rows = 2 * _MAX_DISTANCE - 1
    n_rows = q_len + k_len - 1
    lo_base = q_len - _MAX_DISTANCE

    wid = lax.axis_index("s") * _NUM_CORES + lax.axis_index("c")

    pltpu.sync_copy(tab_hbm, t_vmem)

    hi_row = t_vmem[pl.ds((t_rows - 1) * heads, _LANES)]
    lo_row = t_vmem[pl.ds(0, _LANES)]

    def mid_body(r, carry):
        src = t_vmem[pl.ds(pl.multiple_of(r * heads, heads), _LANES)]
        dst = (lo_base + t_rows - 1) * heads - r * heads
        n_vmem[pl.ds(pl.multiple_of(dst, heads), _LANES)] = src
        return carry

    lax.fori_loop(0, t_rows, mid_body, 0)

    def fill_region(base_words, rows, row_vec):
        def body(it, carry):
            base = base_words + it * (8 * heads)
            for k in range(8):
                n_vmem[pl.ds(pl.multiple_of(base + k * heads, heads), _LANES)] = row_vec
            return carry
        lax.fori_loop(0, rows // 8, body, 0)

    fill_region(0, lo_base, hi_row)
    fill_region((lo_base + t_rows) * heads, n_rows - lo_base - t_rows,
                lo_row)

    row0 = wid * rows_per_w
    row_words = k_len * heads

    def emit(r, carry):
        i = row0 + r
        start = pl.multiple_of((q_len - 1 - i) * heads, heads)
        pltpu.sync_copy(
            n_vmem.at[pl.ds(start, row_words)],
            out_hbm.at[pl.ds(i * row_words, row_words)])
        return carry

    lax.fori_loop(0, rows_per_w, emit, 0)


def kernel(x, relative_attention_bias_table):
    q_len = x.shape[1]
    k_len = x.shape[1]
    t_rows, heads = relative_attention_bias_table.shape
    assert t_rows == 2 * _MAX_DISTANCE - 1 and heads == _LANES
    assert q_len % (_NUM_CORES * _NUM_SUBCORES * 8) == 0

    n_rows_padded = q_len + k_len
    mesh = plsc.VectorSubcoreMesh(core_axis_name="c", subcore_axis_name="s")
    grid_kernel = functools.partial(
        pl.kernel,
        out_type=jax.ShapeDtypeStruct((q_len * k_len * heads,), jnp.float32),
        mesh=mesh,
        scratch_types=[
            pltpu.VMEM((t_rows * heads,), jnp.float32),
            pltpu.VMEM((n_rows_padded * heads,), jnp.float32),
            pltpu.SemaphoreType.DMA,
        ],
    )(functools.partial(_bias_body, q_len, k_len, heads))

    out_flat = grid_kernel(relative_attention_bias_table.reshape(-1))
    return out_flat.reshape(q_len, k_len, heads)
